# baseline (speedup 1.0000x reference)
import functools

import jax
import jax.numpy as jnp
from jax import lax
from jax.experimental import pallas as pl
from jax.experimental.pallas import tpu as pltpu
from jax.experimental.pallas import tpu_sc as plsc

L = 16


def _build_sc_call(batch, fields, cols_per_w):
    mesh = plsc.VectorSubcoreMesh(core_axis_name="c", subcore_axis_name="s")
    ngrp = cols_per_w // L

    @functools.partial(
        pl.kernel,
        mesh=mesh,
        out_type=jax.ShapeDtypeStruct((batch,), jnp.float32),
        compiler_params=pltpu.CompilerParams(needs_layout_passes=False),
        scratch_types=[
            pltpu.VMEM((fields, cols_per_w), jnp.int32),
            pltpu.VMEM((fields,), jnp.float32),
            pltpu.VMEM((fields,), jnp.float32),
            pltpu.VMEM((L,), jnp.float32),
            pltpu.VMEM((cols_per_w,), jnp.float32),
        ],
    )
    def sc_call(xt_hbm, e_hbm, d_hbm, s_hbm, out_hbm, xt_v, e_v, d_v, s_v, out_v):
        wid = lax.axis_index("s") * 2 + lax.axis_index("c")
        base = wid * cols_per_w
        pltpu.sync_copy(e_hbm, e_v)
        pltpu.sync_copy(d_hbm, d_v)
        pltpu.sync_copy(s_hbm, s_v)
        pltpu.sync_copy(xt_hbm.at[:, pl.ds(base, cols_per_w)], xt_v)

        init = s_v[pl.ds(0, L)]

        def field_body(f, accs):
            fidx = jnp.full((L,), f, jnp.int32)
            evec = plsc.load_gather(e_v, [fidx])
            dvec = plsc.load_gather(d_v, [fidx])
            out = []
            for j in range(ngrp):
                u = xt_v[f, pl.ds(j * L, L)].astype(jnp.float32)
                out.append(accs[j] + u * (evec + u * dvec))
            return tuple(out)

        accs = lax.fori_loop(0, fields, field_body, (init,) * ngrp)
        for j in range(ngrp):
            out_v[pl.ds(j * L, L)] = accs[j]
        pltpu.sync_copy(out_v, out_hbm.at[pl.ds(base, cols_per_w)])

    return sc_call


def kernel(x, table, W, b):
    batch, fields = x.shape
    cols_per_w = batch // 32

    w = W.reshape(-1).astype(jnp.float32)
    t0, t1, t2 = table[0], table[1], table[2]
    d = w * ((t2 - (t1 + t1)) + t0) * 0.5
    e = w * (t1 - t0) - d
    s = jnp.full((16,), jnp.sum(w) * t0 + b[0], jnp.float32)

    sc_call = _build_sc_call(batch, fields, cols_per_w)
    return sc_call(x.T, e, d, s).reshape(batch, 1)


# speedup vs baseline: 4.1850x; 4.1850x over previous
import jax
import jax.numpy as jnp
from jax.experimental import pallas as pl
from jax.experimental.pallas import tpu as pltpu

BN = 2048


def _tc_body(tab_ref, b_ref, w_ref, x_ref, o_ref):
    t0, t1, t2 = tab_ref[0], tab_ref[1], tab_ref[2]
    w = w_ref[...]
    d = w * ((t2 - (t1 + t1)) + t0) * 0.5
    e = w * (t1 - t0) - d
    u = x_ref[...].astype(jnp.float32)
    y = u * (e + u * d)
    s = jnp.sum(w) * t0 + b_ref[0]
    o_ref[...] = jnp.sum(y, axis=0) + s


def kernel(x, table, W, b):
    batch, fields = x.shape
    grid = (batch // BN,)
    out = pl.pallas_call(
        _tc_body,
        grid=grid,
        in_specs=[
            pl.BlockSpec(memory_space=pltpu.SMEM),
            pl.BlockSpec(memory_space=pltpu.SMEM),
            pl.BlockSpec((fields, 1), lambda i: (0, 0)),
            pl.BlockSpec((fields, BN), lambda i: (0, i)),
        ],
        out_specs=pl.BlockSpec((BN,), lambda i: (i,)),
        out_shape=jax.ShapeDtypeStruct((batch,), jnp.float32),
        compiler_params=pltpu.CompilerParams(
            dimension_semantics=("parallel",),
        ),
    )(table, b, W, x.T)
    return out.reshape(batch, 1)


# speedup vs baseline: 5.1121x; 1.2215x over previous
import jax
import jax.numpy as jnp
from jax.experimental import pallas as pl
from jax.experimental.pallas import tpu as pltpu

BN = 2048


def _tc_body(tab_ref, b_ref, w_ref, x_ref, o_ref):
    t0, t1, t2 = tab_ref[0], tab_ref[1], tab_ref[2]
    w = w_ref[...]
    d = w * ((t2 - (t1 + t1)) + t0) * 0.5
    e = w * (t1 - t0) - d
    u = x_ref[...].astype(jnp.float32)
    u2 = u * u
    s = jnp.sum(w) * t0 + b_ref[0]
    dn = (((1,), (0,)), ((), ()))
    acc = jax.lax.dot_general(e, u, dn, preferred_element_type=jnp.float32)
    acc += jax.lax.dot_general(d, u2, dn, preferred_element_type=jnp.float32)
    o_ref[...] = acc + s


def kernel(x, table, W, b):
    batch, fields = x.shape
    grid = (batch // BN,)
    out = pl.pallas_call(
        _tc_body,
        grid=grid,
        in_specs=[
            pl.BlockSpec(memory_space=pltpu.SMEM),
            pl.BlockSpec(memory_space=pltpu.SMEM),
            pl.BlockSpec((1, fields), lambda i: (0, 0)),
            pl.BlockSpec((fields, BN), lambda i: (0, i)),
        ],
        out_specs=pl.BlockSpec((1, BN), lambda i: (0, i)),
        out_shape=jax.ShapeDtypeStruct((1, batch), jnp.float32),
        compiler_params=pltpu.CompilerParams(
            dimension_semantics=("parallel",),
        ),
    )(table, b, W.reshape(1, fields), x.T)
    return out.reshape(batch, 1)
